# baseline (speedup 1.0000x reference)
import functools

import jax
import jax.numpy as jnp
from jax import lax
from jax.experimental import pallas as pl
from jax.experimental.pallas import tpu as pltpu
from jax.experimental.pallas import tpu_sc as plsc

N = 10000
E = 320000
D = 128
H1, C1 = 8, 32
HC1 = H1 * C1
C2 = 42
C2P = 48

NPAD = 10240
NW = 32
B = 128
EP = E + N
NBLK = 2592
EPAD = NBLK * B
ROWS_PER_TILE = NPAD // 16
ZCOPIES = ROWS_PER_TILE // B
BLK = 512


def _mesh():
    return plsc.VectorSubcoreMesh(core_axis_name="c", subcore_axis_name="s")


def _tc1_body(x_ref, w1_ref, asrc_ref, adst_ref, ht_ref, as_ref, ad_ref):
    h = jnp.dot(x_ref[...], w1_ref[...], preferred_element_type=jnp.float32)
    ht_ref[0, :, :] = h[:, :128]
    ht_ref[1, :, :] = h[:, 128:]
    als = jnp.dot(h, asrc_ref[...], preferred_element_type=jnp.float32)
    ald = jnp.dot(h, adst_ref[...], preferred_element_type=jnp.float32)
    as_ref[...] = jnp.concatenate([als, als], axis=1)
    ad_ref[...] = jnp.concatenate([ald, ald], axis=1)


def _tc1(x_pad, W1, A_src1, A_dst1):
    grid = (NPAD // BLK,)
    return pl.pallas_call(
        _tc1_body,
        grid=grid,
        in_specs=[
            pl.BlockSpec((BLK, D), lambda i: (i, 0)),
            pl.BlockSpec((D, HC1), lambda i: (0, 0)),
            pl.BlockSpec((HC1, H1), lambda i: (0, 0)),
            pl.BlockSpec((HC1, H1), lambda i: (0, 0)),
        ],
        out_specs=[
            pl.BlockSpec((2, BLK, 128), lambda i: (0, i, 0)),
            pl.BlockSpec((BLK, 16), lambda i: (i, 0)),
            pl.BlockSpec((BLK, 16), lambda i: (i, 0)),
        ],
        out_shape=[
            jax.ShapeDtypeStruct((2, NPAD, 128), jnp.float32),
            jax.ShapeDtypeStruct((NPAD, 16), jnp.float32),
            jax.ShapeDtypeStruct((NPAD, 16), jnp.float32),
        ],
    )(x_pad, W1, A_src1, A_dst1)


def _sc1_body(src_hbm, dst_hbm, as_hbm, ad_hbm, ht_hbm,
              num_out, den_out,
              idxs, idxdA, idxdB, gidx, asr, adrA, adrB,
              eebA, eebB, hrA, hrB,
              nsp, dsp, sem1, sem2, sem3, sem4):
    c = lax.axis_index("c")
    s = lax.axis_index("s")
    base_row = s * ROWS_PER_TILE

    def _zero_row(r, _):
        for j in range(8):
            hrA[r, pl.ds(j * 16, 16)] = jnp.zeros((16,), jnp.float32)
        eebA[r] = jnp.zeros((16,), jnp.float32)
        return _
    lax.fori_loop(0, B // 2, _zero_row, None)
    for k in range(2 * ZCOPIES):
        pltpu.sync_copy(hrA, nsp.at[pl.ds(base_row + k * (B // 2), B // 2)])
        pltpu.sync_copy(eebA, dsp.at[pl.ds(base_row + k * (B // 2), B // 2)])
    plsc.subcore_barrier()

    blocks_per_tile = NBLK // 16
    coff = c * NPAD
    HB = B // 2

    def _edge_block(k, _):
        off = (s * blocks_per_tile + k) * B
        pltpu.sync_copy(src_hbm.at[pl.ds(off, B)], idxs)
        pltpu.sync_copy(dst_hbm.at[pl.ds(off, HB)], idxdA)
        pltpu.sync_copy(dst_hbm.at[pl.ds(off + HB, HB)], idxdB)
        for j in range(8):
            gidx[pl.ds(j * 16, 16)] = idxs[pl.ds(j * 16, 16)] + coff
        cpA1 = pltpu.async_copy(as_hbm.at[idxs], asr, sem1)
        cpA2 = pltpu.async_copy(ad_hbm.at[idxdA], adrA, sem1)
        cpB2 = pltpu.async_copy(ad_hbm.at[idxdB], adrB, sem2)
        cpG1 = pltpu.async_copy(ht_hbm.at[gidx.at[pl.ds(0, HB)]], hrA, sem3)
        cpG2 = pltpu.async_copy(ht_hbm.at[gidx.at[pl.ds(HB, HB)]], hrB, sem4)
        cpA1.wait()
        cpA2.wait()

        def _eeA(r, _):
            e = asr[r] + adrA[r]
            e = jnp.maximum(e, e * 0.2)
            eebA[r] = jnp.exp(e)
            return _
        lax.fori_loop(0, HB, _eeA, None, unroll=2)
        cpG1.wait()

        def _mk_mul(hrX, eebX, base):
            def _mul(r, _):
                v = eebX[r]

                @pl.when(c == 0)
                def _():
                    for j in range(8):
                        m = jnp.full((16,), v[j // 2], jnp.float32)
                        hrX[r, pl.ds(j * 16, 16)] = (
                            hrX[r, pl.ds(j * 16, 16)] * m)

                @pl.when(c == 1)
                def _():
                    for j in range(8):
                        m = jnp.full((16,), v[4 + j // 2], jnp.float32)
                        hrX[r, pl.ds(j * 16, 16)] = (
                            hrX[r, pl.ds(j * 16, 16)] * m)
                return _
            return _mul

        lax.fori_loop(0, HB, _mk_mul(hrA, eebA, 0), None, unroll=2)
        pltpu.sync_copy(hrA, nsp.at[idxdA], add=True)
        pltpu.sync_copy(eebA, dsp.at[idxdA], add=True)

        cpB2.wait()

        def _eeB(r, _):
            e = asr[HB + r] + adrB[r]
            e = jnp.maximum(e, e * 0.2)
            eebB[r] = jnp.exp(e)
            return _
        lax.fori_loop(0, HB, _eeB, None, unroll=2)
        cpG2.wait()
        lax.fori_loop(0, HB, _mk_mul(hrB, eebB, 0), None, unroll=2)
        pltpu.sync_copy(hrB, nsp.at[idxdB], add=True)
        pltpu.sync_copy(eebB, dsp.at[idxdB], add=True)
        return _
    lax.fori_loop(0, blocks_per_tile, _edge_block, None)
    plsc.subcore_barrier()

    for k in range(ZCOPIES):
        r0 = base_row + k * B
        pltpu.sync_copy(nsp.at[pl.ds(r0, B)], num_out.at[pl.ds(coff + r0, B)])
        pltpu.sync_copy(dsp.at[pl.ds(r0, B)], den_out.at[pl.ds(coff + r0, B)])


def _sc1(src_p, dst_p, AS, AD, HT):
    f = pl.kernel(
        _sc1_body,
        out_type=[
            jax.ShapeDtypeStruct((2 * NPAD, 128), jnp.float32),
            jax.ShapeDtypeStruct((2 * NPAD, 16), jnp.float32),
        ],
        mesh=_mesh(),
        compiler_params=pltpu.CompilerParams(use_tc_tiling_on_sc=False),
        scratch_types=[
            pltpu.VMEM((B,), jnp.int32),
            pltpu.VMEM((B // 2,), jnp.int32),
            pltpu.VMEM((B // 2,), jnp.int32),
            pltpu.VMEM((B,), jnp.int32),
            pltpu.VMEM((B, 16), jnp.float32),
            pltpu.VMEM((B // 2, 16), jnp.float32),
            pltpu.VMEM((B // 2, 16), jnp.float32),
            pltpu.VMEM((B // 2, 16), jnp.float32),
            pltpu.VMEM((B // 2, 16), jnp.float32),
            pltpu.VMEM((B // 2, 128), jnp.float32),
            pltpu.VMEM((B // 2, 128), jnp.float32),
            pltpu.VMEM_SHARED((NPAD, 128), jnp.float32),
            pltpu.VMEM_SHARED((NPAD, 16), jnp.float32),
            pltpu.SemaphoreType.DMA,
            pltpu.SemaphoreType.DMA,
            pltpu.SemaphoreType.DMA,
            pltpu.SemaphoreType.DMA,
        ],
    )
    return f(src_p, dst_p, AS, AD, HT)


def _tc2_body(n1_ref, d1_ref, rexp_ref, w2_ref, asp_ref, adp_ref, b1_ref,
              h2_ref, as2_ref, ad2_ref):
    ncat = jnp.concatenate([n1_ref[0, :, :], n1_ref[1, :, :]], axis=1)
    d8 = d1_ref[...][:, :8]
    dfull = jnp.dot(d8, rexp_ref[...], preferred_element_type=jnp.float32)
    o = ncat / (dfull + 1e-16) + b1_ref[...]
    o = jnp.maximum(o, 0.0)
    h2 = jnp.dot(o, w2_ref[...], preferred_element_type=jnp.float32)
    h2_ref[...] = h2
    als = jnp.dot(h2, asp_ref[...], preferred_element_type=jnp.float32)
    ald = jnp.dot(h2, adp_ref[...], preferred_element_type=jnp.float32)
    as2_ref[...] = jnp.broadcast_to(als, (als.shape[0], 16))
    ad2_ref[...] = jnp.broadcast_to(ald, (ald.shape[0], 16))


def _tc2(n1, d1, Rexp, W2p, asp2, adp2, b1r):
    grid = (NPAD // BLK,)
    return pl.pallas_call(
        _tc2_body,
        grid=grid,
        in_specs=[
            pl.BlockSpec((2, BLK, 128), lambda i: (0, i, 0)),
            pl.BlockSpec((BLK, 16), lambda i: (i, 0)),
            pl.BlockSpec((H1, HC1), lambda i: (0, 0)),
            pl.BlockSpec((HC1, C2P), lambda i: (0, 0)),
            pl.BlockSpec((C2P, 1), lambda i: (0, 0)),
            pl.BlockSpec((C2P, 1), lambda i: (0, 0)),
            pl.BlockSpec((1, HC1), lambda i: (0, 0)),
        ],
        out_specs=[
            pl.BlockSpec((BLK, C2P), lambda i: (i, 0)),
            pl.BlockSpec((BLK, 16), lambda i: (i, 0)),
            pl.BlockSpec((BLK, 16), lambda i: (i, 0)),
        ],
        out_shape=[
            jax.ShapeDtypeStruct((NPAD, C2P), jnp.float32),
            jax.ShapeDtypeStruct((NPAD, 16), jnp.float32),
            jax.ShapeDtypeStruct((NPAD, 16), jnp.float32),
        ],
    )(n1, d1, Rexp, W2p, asp2, adp2, b1r)


def _sc2_body(src_hbm, dst_hbm, as_hbm, ad_hbm, ht_hbm,
              num_out, den_out,
              idxs, idxd, asr, adr, eeb, hr, zb, zbd,
              nsp, dsp, sem1, sem2, sem3):
    c = lax.axis_index("c")
    s = lax.axis_index("s")
    base_row = s * ROWS_PER_TILE

    def _zero_row(r, _):
        for j in range(3):
            zb[r, pl.ds(j * 16, 16)] = jnp.zeros((16,), jnp.float32)
        zbd[r] = jnp.zeros((16,), jnp.float32)
        return _
    lax.fori_loop(0, B, _zero_row, None)
    for k in range(ZCOPIES):
        pltpu.sync_copy(zb, nsp.at[pl.ds(base_row + k * B, B)])
        pltpu.sync_copy(zbd, dsp.at[pl.ds(base_row + k * B, B)])
    plsc.subcore_barrier()

    w = s * 2 + c
    blocks_per_worker = NBLK // NW
    coff = c * NPAD

    def _edge_block(k, _):
        off = (w * blocks_per_worker + k) * B
        pltpu.sync_copy(src_hbm.at[pl.ds(off, B)], idxs)
        pltpu.sync_copy(dst_hbm.at[pl.ds(off, B)], idxd)
        cp1 = pltpu.async_copy(as_hbm.at[idxs], asr, sem1)
        cp2 = pltpu.async_copy(ad_hbm.at[idxd], adr, sem2)
        cp3 = pltpu.async_copy(ht_hbm.at[idxs], hr, sem3)
        cp1.wait()
        cp2.wait()

        def _ee(r, _):
            e = asr[r] + adr[r]
            e = jnp.maximum(e, e * 0.2)
            eeb[r] = jnp.exp(e)
            return _
        lax.fori_loop(0, B, _ee, None)
        cp3.wait()

        def _mul(r, _):
            sc = eeb[r]
            for j in range(3):
                hr[r, pl.ds(j * 16, 16)] = hr[r, pl.ds(j * 16, 16)] * sc
            return _
        lax.fori_loop(0, B, _mul, None)

        pltpu.sync_copy(hr, nsp.at[idxd], add=True)
        pltpu.sync_copy(eeb, dsp.at[idxd], add=True)
        return _
    lax.fori_loop(0, blocks_per_worker, _edge_block, None)
    plsc.subcore_barrier()

    for k in range(ZCOPIES):
        r0 = base_row + k * B
        pltpu.sync_copy(nsp.at[pl.ds(r0, B)], num_out.at[pl.ds(coff + r0, B)])
        pltpu.sync_copy(dsp.at[pl.ds(r0, B)], den_out.at[pl.ds(coff + r0, B)])


def _sc2(src_p, dst_p, AS2, AD2, H2T):
    f = pl.kernel(
        _sc2_body,
        out_type=[
            jax.ShapeDtypeStruct((2 * NPAD, C2P), jnp.float32),
            jax.ShapeDtypeStruct((2 * NPAD, 16), jnp.float32),
        ],
        mesh=_mesh(),
        compiler_params=pltpu.CompilerParams(use_tc_tiling_on_sc=False),
        scratch_types=[
            pltpu.VMEM((B,), jnp.int32),
            pltpu.VMEM((B,), jnp.int32),
            pltpu.VMEM((B, 16), jnp.float32),
            pltpu.VMEM((B, 16), jnp.float32),
            pltpu.VMEM((B, 16), jnp.float32),
            pltpu.VMEM((B, C2P), jnp.float32),
            pltpu.VMEM((B, C2P), jnp.float32),
            pltpu.VMEM((B, 16), jnp.float32),
            pltpu.VMEM_SHARED((NPAD, C2P), jnp.float32),
            pltpu.VMEM_SHARED((NPAD, 16), jnp.float32),
            pltpu.SemaphoreType.DMA,
            pltpu.SemaphoreType.DMA,
            pltpu.SemaphoreType.DMA,
        ],
    )
    return f(src_p, dst_p, AS2, AD2, H2T)


def _tc3_body(n2_ref, d2_ref, b2_ref, out_ref):
    nsum = n2_ref[0, :, :] + n2_ref[1, :, :]
    dsum = d2_ref[0, :, :1] + d2_ref[1, :, :1]
    o = nsum / (dsum + 1e-16) + b2_ref[...]
    mask = lax.broadcasted_iota(jnp.int32, o.shape, 1) < C2
    om = jnp.where(mask, o, -1e30)
    m = jnp.max(om, axis=1, keepdims=True)
    ex = jnp.where(mask, jnp.exp(o - m), 0.0)
    lse = m + jnp.log(jnp.sum(ex, axis=1, keepdims=True))
    out_ref[...] = o - lse


def _tc3(n2, d2, b2r):
    grid = (NPAD // BLK,)
    return pl.pallas_call(
        _tc3_body,
        grid=grid,
        in_specs=[
            pl.BlockSpec((2, BLK, C2P), lambda i: (0, i, 0)),
            pl.BlockSpec((2, BLK, 16), lambda i: (0, i, 0)),
            pl.BlockSpec((1, C2P), lambda i: (0, 0)),
        ],
        out_specs=pl.BlockSpec((BLK, C2P), lambda i: (i, 0)),
        out_shape=jax.ShapeDtypeStruct((NPAD, C2P), jnp.float32),
    )(n2, d2, b2r)


def kernel(x, edge_index, W1, a_src1, a_dst1, b1, W2, a_src2, a_dst2, b2):
    f32 = jnp.float32
    x_pad = jnp.zeros((NPAD, D), f32).at[:N].set(x)
    loops = jnp.arange(N, dtype=jnp.int32)
    src_p = jnp.full((EPAD,), N, jnp.int32)
    src_p = src_p.at[:E].set(edge_index[0]).at[E:EP].set(loops)
    dst_p = jnp.full((EPAD,), N, jnp.int32)
    dst_p = dst_p.at[:E].set(edge_index[1]).at[E:EP].set(loops)

    eye8 = jnp.eye(H1, dtype=f32)
    A_src1 = (eye8[:, None, :] * a_src1[:, :, None]).reshape(HC1, H1)
    A_dst1 = (eye8[:, None, :] * a_dst1[:, :, None]).reshape(HC1, H1)
    Rexp = (eye8[:, :, None] * jnp.ones((1, 1, C1), f32)).reshape(H1, HC1)
    W2p = jnp.zeros((HC1, C2P), f32).at[:, :C2].set(W2)
    asp2 = jnp.zeros((C2P, 1), f32).at[:C2, 0].set(a_src2[0])
    adp2 = jnp.zeros((C2P, 1), f32).at[:C2, 0].set(a_dst2[0])
    b1r = b1.reshape(1, HC1).astype(f32)
    b2r = jnp.zeros((1, C2P), f32).at[0, :C2].set(b2)

    HT, AS, AD = _tc1(x_pad, W1, A_src1, A_dst1)
    HTf = HT.reshape(2 * NPAD, 128)
    n1, d1 = _sc1(src_p, dst_p, AS, AD, HTf)
    n1 = n1.reshape(2, NPAD, 128)
    d1 = d1[:NPAD]

    H2T, AS2, AD2 = _tc2(n1, d1, Rexp, W2p, asp2, adp2, b1r)
    n2, d2 = _sc2(src_p, dst_p, AS2, AD2, H2T)
    out = _tc3(n2.reshape(2, NPAD, C2P), d2.reshape(2, NPAD, 16), b2r)
    return out[:N, :C2]


# speedup vs baseline: 1.0916x; 1.0916x over previous
import functools

import jax
import jax.numpy as jnp
from jax import lax
from jax.experimental import pallas as pl
from jax.experimental.pallas import tpu as pltpu
from jax.experimental.pallas import tpu_sc as plsc

N = 10000
E = 320000
D = 128
H1, C1 = 8, 32
HC1 = H1 * C1
C2 = 42
C2P = 48

NPAD = 10240
NW = 32
B = 128
EP = E + N
NBLK = 2592
EPAD = NBLK * B
ROWS_PER_TILE = NPAD // 16
ZCOPIES = ROWS_PER_TILE // B
BLK = 512


def _mesh():
    return plsc.VectorSubcoreMesh(core_axis_name="c", subcore_axis_name="s")


def _tc1_body(x_ref, w1_ref, asrc_ref, adst_ref, ht_ref, as_ref, ad_ref):
    h = jnp.dot(x_ref[...], w1_ref[...], preferred_element_type=jnp.float32)
    ht_ref[0, :, :] = h[:, :128]
    ht_ref[1, :, :] = h[:, 128:]
    als = jnp.dot(h, asrc_ref[...], preferred_element_type=jnp.float32)
    ald = jnp.dot(h, adst_ref[...], preferred_element_type=jnp.float32)
    as_ref[...] = jnp.concatenate([als, als], axis=1)
    ad_ref[...] = jnp.concatenate([ald, ald], axis=1)


def _tc1(x_pad, W1, A_src1, A_dst1):
    grid = (NPAD // BLK,)
    return pl.pallas_call(
        _tc1_body,
        grid=grid,
        in_specs=[
            pl.BlockSpec((BLK, D), lambda i: (i, 0)),
            pl.BlockSpec((D, HC1), lambda i: (0, 0)),
            pl.BlockSpec((HC1, H1), lambda i: (0, 0)),
            pl.BlockSpec((HC1, H1), lambda i: (0, 0)),
        ],
        out_specs=[
            pl.BlockSpec((2, BLK, 128), lambda i: (0, i, 0)),
            pl.BlockSpec((BLK, 16), lambda i: (i, 0)),
            pl.BlockSpec((BLK, 16), lambda i: (i, 0)),
        ],
        out_shape=[
            jax.ShapeDtypeStruct((2, NPAD, 128), jnp.float32),
            jax.ShapeDtypeStruct((NPAD, 16), jnp.float32),
            jax.ShapeDtypeStruct((NPAD, 16), jnp.float32),
        ],
    )(x_pad, W1, A_src1, A_dst1)


def _sc1_body(src_hbm, dst_hbm, as_hbm, ad_hbm, ht_hbm,
              num_out, den_out,
              idxs, idxd, gidx, asr, adr, eeb, hr,
              nsp, dsp, sem1, sem2, sem3):
    c = lax.axis_index("c")
    s = lax.axis_index("s")
    base_row = s * ROWS_PER_TILE

    def _zero_row(r, _):
        for j in range(8):
            hr[r, pl.ds(j * 16, 16)] = jnp.zeros((16,), jnp.float32)
        eeb[r] = jnp.zeros((16,), jnp.float32)
        return _
    lax.fori_loop(0, B, _zero_row, None)
    for k in range(ZCOPIES):
        pltpu.sync_copy(hr, nsp.at[pl.ds(base_row + k * B, B)])
        pltpu.sync_copy(eeb, dsp.at[pl.ds(base_row + k * B, B)])
    plsc.subcore_barrier()

    blocks_per_tile = NBLK // 16
    coff = c * NPAD
    HB = B // 2

    def _edge_block(k, _):
        off = (s * blocks_per_tile + k) * B
        pltpu.sync_copy(src_hbm.at[pl.ds(off, B)], idxs)
        pltpu.sync_copy(dst_hbm.at[pl.ds(off, B)], idxd)
        for j in range(8):
            gidx[pl.ds(j * 16, 16)] = idxs[pl.ds(j * 16, 16)] + coff
        cp1 = pltpu.async_copy(as_hbm.at[idxs], asr, sem1)
        cp2 = pltpu.async_copy(ad_hbm.at[idxd], adr, sem2)
        cp3 = pltpu.async_copy(ht_hbm.at[gidx], hr, sem3)
        cp1.wait()
        cp2.wait()

        def _ee(r, _):
            e = asr[r] + adr[r]
            e = jnp.maximum(e, e * 0.2)
            eeb[r] = jnp.exp(e)
            return _
        lax.fori_loop(0, B, _ee, None, unroll=2)
        cp3.wait()

        def _mul(r, _):
            v = eeb[r]

            @pl.when(c == 0)
            def _():
                for j in range(8):
                    m = jnp.full((16,), v[j // 2], jnp.float32)
                    hr[r, pl.ds(j * 16, 16)] = hr[r, pl.ds(j * 16, 16)] * m

            @pl.when(c == 1)
            def _():
                for j in range(8):
                    m = jnp.full((16,), v[4 + j // 2], jnp.float32)
                    hr[r, pl.ds(j * 16, 16)] = hr[r, pl.ds(j * 16, 16)] * m
            return _
        lax.fori_loop(0, B, _mul, None, unroll=2)

        pltpu.sync_copy(hr, nsp.at[idxd], add=True)
        pltpu.sync_copy(eeb, dsp.at[idxd], add=True)
        return _
    lax.fori_loop(0, blocks_per_tile, _edge_block, None)
    plsc.subcore_barrier()

    for k in range(ZCOPIES):
        r0 = base_row + k * B
        pltpu.sync_copy(nsp.at[pl.ds(r0, B)], num_out.at[pl.ds(coff + r0, B)])
        pltpu.sync_copy(dsp.at[pl.ds(r0, B)], den_out.at[pl.ds(coff + r0, B)])


def _sc1(src_p, dst_p, AS, AD, HT):
    f = pl.kernel(
        _sc1_body,
        out_type=[
            jax.ShapeDtypeStruct((2 * NPAD, 128), jnp.float32),
            jax.ShapeDtypeStruct((2 * NPAD, 16), jnp.float32),
        ],
        mesh=_mesh(),
        compiler_params=pltpu.CompilerParams(use_tc_tiling_on_sc=False),
        scratch_types=[
            pltpu.VMEM((B,), jnp.int32),
            pltpu.VMEM((B,), jnp.int32),
            pltpu.VMEM((B,), jnp.int32),
            pltpu.VMEM((B, 16), jnp.float32),
            pltpu.VMEM((B, 16), jnp.float32),
            pltpu.VMEM((B, 16), jnp.float32),
            pltpu.VMEM((B, 128), jnp.float32),
            pltpu.VMEM_SHARED((NPAD, 128), jnp.float32),
            pltpu.VMEM_SHARED((NPAD, 16), jnp.float32),
            pltpu.SemaphoreType.DMA,
            pltpu.SemaphoreType.DMA,
            pltpu.SemaphoreType.DMA,
        ],
    )
    return f(src_p, dst_p, AS, AD, HT)


def _tc2_body(n1_ref, d1_ref, rexp_ref, w2_ref, asp_ref, adp_ref, b1_ref,
              h2_ref, as2_ref, ad2_ref):
    ncat = jnp.concatenate([n1_ref[0, :, :], n1_ref[1, :, :]], axis=1)
    d8 = d1_ref[...][:, :8]
    dfull = jnp.dot(d8, rexp_ref[...], preferred_element_type=jnp.float32)
    o = ncat / (dfull + 1e-16) + b1_ref[...]
    o = jnp.maximum(o, 0.0)
    h2 = jnp.dot(o, w2_ref[...], preferred_element_type=jnp.float32)
    h2_ref[...] = h2
    als = jnp.dot(h2, asp_ref[...], preferred_element_type=jnp.float32)
    ald = jnp.dot(h2, adp_ref[...], preferred_element_type=jnp.float32)
    as2_ref[...] = jnp.broadcast_to(als, (als.shape[0], 16))
    ad2_ref[...] = jnp.broadcast_to(ald, (ald.shape[0], 16))


def _tc2(n1, d1, Rexp, W2p, asp2, adp2, b1r):
    grid = (NPAD // BLK,)
    return pl.pallas_call(
        _tc2_body,
        grid=grid,
        in_specs=[
            pl.BlockSpec((2, BLK, 128), lambda i: (0, i, 0)),
            pl.BlockSpec((BLK, 16), lambda i: (i, 0)),
            pl.BlockSpec((H1, HC1), lambda i: (0, 0)),
            pl.BlockSpec((HC1, C2P), lambda i: (0, 0)),
            pl.BlockSpec((C2P, 1), lambda i: (0, 0)),
            pl.BlockSpec((C2P, 1), lambda i: (0, 0)),
            pl.BlockSpec((1, HC1), lambda i: (0, 0)),
        ],
        out_specs=[
            pl.BlockSpec((BLK, C2P), lambda i: (i, 0)),
            pl.BlockSpec((BLK, 16), lambda i: (i, 0)),
            pl.BlockSpec((BLK, 16), lambda i: (i, 0)),
        ],
        out_shape=[
            jax.ShapeDtypeStruct((NPAD, C2P), jnp.float32),
            jax.ShapeDtypeStruct((NPAD, 16), jnp.float32),
            jax.ShapeDtypeStruct((NPAD, 16), jnp.float32),
        ],
    )(n1, d1, Rexp, W2p, asp2, adp2, b1r)


def _sc2_body(src_hbm, dst_hbm, as_hbm, ad_hbm, ht_hbm,
              num_out, den_out,
              idxs, idxd, asr, adr, eeb, hr, zb, zbd,
              nsp, dsp, sem1, sem2, sem3):
    c = lax.axis_index("c")
    s = lax.axis_index("s")
    base_row = s * ROWS_PER_TILE

    def _zero_row(r, _):
        for j in range(3):
            zb[r, pl.ds(j * 16, 16)] = jnp.zeros((16,), jnp.float32)
        zbd[r] = jnp.zeros((16,), jnp.float32)
        return _
    lax.fori_loop(0, B, _zero_row, None)
    for k in range(ZCOPIES):
        pltpu.sync_copy(zb, nsp.at[pl.ds(base_row + k * B, B)])
        pltpu.sync_copy(zbd, dsp.at[pl.ds(base_row + k * B, B)])
    plsc.subcore_barrier()

    w = s * 2 + c
    blocks_per_worker = NBLK // NW
    coff = c * NPAD

    def _edge_block(k, _):
        off = (w * blocks_per_worker + k) * B
        pltpu.sync_copy(src_hbm.at[pl.ds(off, B)], idxs)
        pltpu.sync_copy(dst_hbm.at[pl.ds(off, B)], idxd)
        cp1 = pltpu.async_copy(as_hbm.at[idxs], asr, sem1)
        cp2 = pltpu.async_copy(ad_hbm.at[idxd], adr, sem2)
        cp3 = pltpu.async_copy(ht_hbm.at[idxs], hr, sem3)
        cp1.wait()
        cp2.wait()

        def _ee(r, _):
            e = asr[r] + adr[r]
            e = jnp.maximum(e, e * 0.2)
            eeb[r] = jnp.exp(e)
            return _
        lax.fori_loop(0, B, _ee, None)
        cp3.wait()

        def _mul(r, _):
            sc = eeb[r]
            for j in range(3):
                hr[r, pl.ds(j * 16, 16)] = hr[r, pl.ds(j * 16, 16)] * sc
            return _
        lax.fori_loop(0, B, _mul, None)

        pltpu.sync_copy(hr, nsp.at[idxd], add=True)
        pltpu.sync_copy(eeb, dsp.at[idxd], add=True)
        return _
    lax.fori_loop(0, blocks_per_worker, _edge_block, None)
    plsc.subcore_barrier()

    for k in range(ZCOPIES):
        r0 = base_row + k * B
        pltpu.sync_copy(nsp.at[pl.ds(r0, B)], num_out.at[pl.ds(coff + r0, B)])
        pltpu.sync_copy(dsp.at[pl.ds(r0, B)], den_out.at[pl.ds(coff + r0, B)])


def _sc2(src_p, dst_p, AS2, AD2, H2T):
    f = pl.kernel(
        _sc2_body,
        out_type=[
            jax.ShapeDtypeStruct((2 * NPAD, C2P), jnp.float32),
            jax.ShapeDtypeStruct((2 * NPAD, 16), jnp.float32),
        ],
        mesh=_mesh(),
        compiler_params=pltpu.CompilerParams(use_tc_tiling_on_sc=False),
        scratch_types=[
            pltpu.VMEM((B,), jnp.int32),
            pltpu.VMEM((B,), jnp.int32),
            pltpu.VMEM((B, 16), jnp.float32),
            pltpu.VMEM((B, 16), jnp.float32),
            pltpu.VMEM((B, 16), jnp.float32),
            pltpu.VMEM((B, C2P), jnp.float32),
            pltpu.VMEM((B, C2P), jnp.float32),
            pltpu.VMEM((B, 16), jnp.float32),
            pltpu.VMEM_SHARED((NPAD, C2P), jnp.float32),
            pltpu.VMEM_SHARED((NPAD, 16), jnp.float32),
            pltpu.SemaphoreType.DMA,
            pltpu.SemaphoreType.DMA,
            pltpu.SemaphoreType.DMA,
        ],
    )
    return f(src_p, dst_p, AS2, AD2, H2T)


def _tc3_body(n2_ref, d2_ref, b2_ref, out_ref):
    nsum = n2_ref[0, :, :] + n2_ref[1, :, :]
    dsum = d2_ref[0, :, :1] + d2_ref[1, :, :1]
    o = nsum / (dsum + 1e-16) + b2_ref[...]
    mask = lax.broadcasted_iota(jnp.int32, o.shape, 1) < C2
    om = jnp.where(mask, o, -1e30)
    m = jnp.max(om, axis=1, keepdims=True)
    ex = jnp.where(mask, jnp.exp(o - m), 0.0)
    lse = m + jnp.log(jnp.sum(ex, axis=1, keepdims=True))
    out_ref[...] = o - lse


def _tc3(n2, d2, b2r):
    grid = (NPAD // BLK,)
    return pl.pallas_call(
        _tc3_body,
        grid=grid,
        in_specs=[
            pl.BlockSpec((2, BLK, C2P), lambda i: (0, i, 0)),
            pl.BlockSpec((2, BLK, 16), lambda i: (0, i, 0)),
            pl.BlockSpec((1, C2P), lambda i: (0, 0)),
        ],
        out_specs=pl.BlockSpec((BLK, C2P), lambda i: (i, 0)),
        out_shape=jax.ShapeDtypeStruct((NPAD, C2P), jnp.float32),
    )(n2, d2, b2r)


def kernel(x, edge_index, W1, a_src1, a_dst1, b1, W2, a_src2, a_dst2, b2):
    f32 = jnp.float32
    x_pad = jnp.zeros((NPAD, D), f32).at[:N].set(x)
    loops = jnp.arange(N, dtype=jnp.int32)
    src_p = jnp.full((EPAD,), N, jnp.int32)
    src_p = src_p.at[:E].set(edge_index[0]).at[E:EP].set(loops)
    dst_p = jnp.full((EPAD,), N, jnp.int32)
    dst_p = dst_p.at[:E].set(edge_index[1]).at[E:EP].set(loops)

    eye8 = jnp.eye(H1, dtype=f32)
    A_src1 = (eye8[:, None, :] * a_src1[:, :, None]).reshape(HC1, H1)
    A_dst1 = (eye8[:, None, :] * a_dst1[:, :, None]).reshape(HC1, H1)
    Rexp = (eye8[:, :, None] * jnp.ones((1, 1, C1), f32)).reshape(H1, HC1)
    W2p = jnp.zeros((HC1, C2P), f32).at[:, :C2].set(W2)
    asp2 = jnp.zeros((C2P, 1), f32).at[:C2, 0].set(a_src2[0])
    adp2 = jnp.zeros((C2P, 1), f32).at[:C2, 0].set(a_dst2[0])
    b1r = b1.reshape(1, HC1).astype(f32)
    b2r = jnp.zeros((1, C2P), f32).at[0, :C2].set(b2)

    HT, AS, AD = _tc1(x_pad, W1, A_src1, A_dst1)
    HTf = HT.reshape(2 * NPAD, 128)
    n1, d1 = _sc1(src_p, dst_p, AS, AD, HTf)
    n1 = n1.reshape(2, NPAD, 128)
    d1 = d1[:NPAD]

    H2T, AS2, AD2 = _tc2(n1, d1, Rexp, W2p, asp2, adp2, b1r)
    n2, d2 = _sc2(src_p, dst_p, AS2, AD2, H2T)
    out = _tc3(n2.reshape(2, NPAD, C2P), d2.reshape(2, NPAD, 16), b2r)
    return out[:N, :C2]


# speedup vs baseline: 1.3048x; 1.1954x over previous
import jax
import jax.numpy as jnp
from jax import lax
from jax.experimental import pallas as pl
from jax.experimental.pallas import tpu as pltpu
from jax.experimental.pallas import tpu_sc as plsc

N = 10000
E = 320000
D = 128
H1, C1 = 8, 32
HC1 = H1 * C1
C2 = 42
C2P = 48

NPAD = 10240
NW = 32
B = 128
EP = E + N
NBLK = 2592
EPAD = NBLK * B
ROWS_PER_TILE = NPAD // 16
ZCOPIES = ROWS_PER_TILE // B
BLK = 512


def _mesh():
    return plsc.VectorSubcoreMesh(core_axis_name="c", subcore_axis_name="s")


def _tc1_body(x_ref, w1_ref, asrc_ref, adst_ref, ht_ref, as_ref, ad_ref):
    h = jnp.dot(x_ref[...], w1_ref[...], preferred_element_type=jnp.float32)
    ht_ref[0, :, :] = h[:, :128]
    ht_ref[1, :, :] = h[:, 128:]
    als = jnp.dot(h, asrc_ref[...], preferred_element_type=jnp.float32)
    ald = jnp.dot(h, adst_ref[...], preferred_element_type=jnp.float32)
    as_ref[...] = jnp.concatenate([als, als], axis=1)
    ad_ref[...] = jnp.concatenate([ald, ald], axis=1)


def _tc1(x_pad, W1, A_src1, A_dst1):
    grid = (NPAD // BLK,)
    return pl.pallas_call(
        _tc1_body,
        grid=grid,
        in_specs=[
            pl.BlockSpec((BLK, D), lambda i: (i, 0)),
            pl.BlockSpec((D, HC1), lambda i: (0, 0)),
            pl.BlockSpec((HC1, H1), lambda i: (0, 0)),
            pl.BlockSpec((HC1, H1), lambda i: (0, 0)),
        ],
        out_specs=[
            pl.BlockSpec((2, BLK, 128), lambda i: (0, i, 0)),
            pl.BlockSpec((BLK, 16), lambda i: (i, 0)),
            pl.BlockSpec((BLK, 16), lambda i: (i, 0)),
        ],
        out_shape=[
            jax.ShapeDtypeStruct((2, NPAD, 128), jnp.float32),
            jax.ShapeDtypeStruct((NPAD, 16), jnp.float32),
            jax.ShapeDtypeStruct((NPAD, 16), jnp.float32),
        ],
    )(x_pad, W1, A_src1, A_dst1)


def _sc1_body(src_hbm, dst_hbm, as_hbm, ad_hbm, ht_hbm,
              num_out, den_out,
              idxs, idxd, gidx, asr, adr, eeb, hr,
              nsp, dsp, sem1, sem2, sem3):
    c = lax.axis_index("c")
    s = lax.axis_index("s")
    base_row = s * ROWS_PER_TILE

    def _zero_row(r, _):
        for j in range(8):
            hr[r, pl.ds(j * 16, 16)] = jnp.zeros((16,), jnp.float32)
        eeb[r] = jnp.zeros((16,), jnp.float32)
        return _
    lax.fori_loop(0, B, _zero_row, None)
    for k in range(ZCOPIES):
        pltpu.sync_copy(hr, nsp.at[pl.ds(base_row + k * B, B)])
        pltpu.sync_copy(eeb, dsp.at[pl.ds(base_row + k * B, B)])
    plsc.subcore_barrier()

    blocks_per_tile = NBLK // 16
    coff = c * NPAD

    def _edge_block(k, _):
        off = (s * blocks_per_tile + k) * B
        pltpu.sync_copy(src_hbm.at[pl.ds(off, B)], idxs)
        pltpu.sync_copy(dst_hbm.at[pl.ds(off, B)], idxd)
        for j in range(8):
            gidx[pl.ds(j * 16, 16)] = idxs[pl.ds(j * 16, 16)] + coff
        cp1 = pltpu.async_copy(as_hbm.at[idxs], asr, sem1)
        cp2 = pltpu.async_copy(ad_hbm.at[idxd], adr, sem2)
        cp3 = pltpu.async_copy(ht_hbm.at[gidx], hr, sem3)
        cp1.wait()
        cp2.wait()

        def _ee(r, _):
            e = asr[r] + adr[r]
            e = jnp.maximum(e, e * 0.2)
            eeb[r] = jnp.exp(e)
            return _
        lax.fori_loop(0, B, _ee, None)
        cp3.wait()

        def _mul(r, _):
            v = eeb[r]

            @pl.when(c == 0)
            def _():
                for j in range(8):
                    m = jnp.full((16,), v[j // 2], jnp.float32)
                    hr[r, pl.ds(j * 16, 16)] = hr[r, pl.ds(j * 16, 16)] * m

            @pl.when(c == 1)
            def _():
                for j in range(8):
                    m = jnp.full((16,), v[4 + j // 2], jnp.float32)
                    hr[r, pl.ds(j * 16, 16)] = hr[r, pl.ds(j * 16, 16)] * m
            return _
        lax.fori_loop(0, B, _mul, None)

        pltpu.sync_copy(hr, nsp.at[idxd], add=True)
        pltpu.sync_copy(eeb, dsp.at[idxd], add=True)
        return _
    lax.fori_loop(0, blocks_per_tile, _edge_block, None)
    plsc.subcore_barrier()

    for k in range(ZCOPIES):
        r0 = base_row + k * B
        pltpu.sync_copy(nsp.at[pl.ds(r0, B)], num_out.at[pl.ds(coff + r0, B)])
        pltpu.sync_copy(dsp.at[pl.ds(r0, B)], den_out.at[pl.ds(coff + r0, B)])


def _sc1(src_p, dst_p, AS, AD, HT):
    f = pl.kernel(
        _sc1_body,
        out_type=[
            jax.ShapeDtypeStruct((2 * NPAD, 128), jnp.float32),
            jax.ShapeDtypeStruct((2 * NPAD, 16), jnp.float32),
        ],
        mesh=_mesh(),
        compiler_params=pltpu.CompilerParams(use_tc_tiling_on_sc=False),
        scratch_types=[
            pltpu.VMEM((B,), jnp.int32),
            pltpu.VMEM((B,), jnp.int32),
            pltpu.VMEM((B,), jnp.int32),
            pltpu.VMEM((B, 16), jnp.float32),
            pltpu.VMEM((B, 16), jnp.float32),
            pltpu.VMEM((B, 16), jnp.float32),
            pltpu.VMEM((B, 128), jnp.float32),
            pltpu.VMEM_SHARED((NPAD, 128), jnp.float32),
            pltpu.VMEM_SHARED((NPAD, 16), jnp.float32),
            pltpu.SemaphoreType.DMA,
            pltpu.SemaphoreType.DMA,
            pltpu.SemaphoreType.DMA,
        ],
    )
    return f(src_p, dst_p, AS, AD, HT)


def _tc2_body(n1_ref, d1_ref, rexp_ref, w2_ref, asp_ref, adp_ref, b1_ref,
              h2_ref, as2_ref, ad2_ref):
    ncat = jnp.concatenate([n1_ref[0, :, :], n1_ref[1, :, :]], axis=1)
    d8 = d1_ref[...][:, :8]
    dfull = jnp.dot(d8, rexp_ref[...], preferred_element_type=jnp.float32)
    o = ncat / (dfull + 1e-16) + b1_ref[...]
    o = jnp.maximum(o, 0.0)
    h2 = jnp.dot(o, w2_ref[...], preferred_element_type=jnp.float32)
    h2_ref[...] = h2
    als = jnp.dot(h2, asp_ref[...], preferred_element_type=jnp.float32)
    ald = jnp.dot(h2, adp_ref[...], preferred_element_type=jnp.float32)
    as2_ref[...] = jnp.broadcast_to(als, (als.shape[0], 16))
    ad2_ref[...] = jnp.broadcast_to(ald, (ald.shape[0], 16))


def _tc2(n1, d1, Rexp, W2p, asp2, adp2, b1r):
    grid = (NPAD // BLK,)
    return pl.pallas_call(
        _tc2_body,
        grid=grid,
        in_specs=[
            pl.BlockSpec((2, BLK, 128), lambda i: (0, i, 0)),
            pl.BlockSpec((BLK, 16), lambda i: (i, 0)),
            pl.BlockSpec((H1, HC1), lambda i: (0, 0)),
            pl.BlockSpec((HC1, C2P), lambda i: (0, 0)),
            pl.BlockSpec((C2P, 1), lambda i: (0, 0)),
            pl.BlockSpec((C2P, 1), lambda i: (0, 0)),
            pl.BlockSpec((1, HC1), lambda i: (0, 0)),
        ],
        out_specs=[
            pl.BlockSpec((BLK, C2P), lambda i: (i, 0)),
            pl.BlockSpec((BLK, 16), lambda i: (i, 0)),
            pl.BlockSpec((BLK, 16), lambda i: (i, 0)),
        ],
        out_shape=[
            jax.ShapeDtypeStruct((NPAD, C2P), jnp.float32),
            jax.ShapeDtypeStruct((NPAD, 16), jnp.float32),
            jax.ShapeDtypeStruct((NPAD, 16), jnp.float32),
        ],
    )(n1, d1, Rexp, W2p, asp2, adp2, b1r)


def _sc2_body(src_hbm, dst_hbm, as_hbm, ad_hbm, ht_hbm,
              num_out, den_out,
              idxs, idxd, asr, adr, eeb, hr, zb, zbd,
              nsp, dsp, sem1, sem2, sem3):
    c = lax.axis_index("c")
    s = lax.axis_index("s")
    base_row = s * ROWS_PER_TILE

    def _zero_row(r, _):
        for j in range(3):
            zb[r, pl.ds(j * 16, 16)] = jnp.zeros((16,), jnp.float32)
        zbd[r] = jnp.zeros((16,), jnp.float32)
        return _
    lax.fori_loop(0, B, _zero_row, None)
    for k in range(ZCOPIES):
        pltpu.sync_copy(zb, nsp.at[pl.ds(base_row + k * B, B)])
        pltpu.sync_copy(zbd, dsp.at[pl.ds(base_row + k * B, B)])
    plsc.subcore_barrier()

    w = s * 2 + c
    blocks_per_worker = NBLK // NW
    coff = c * NPAD

    def _edge_block(k, _):
        off = (w * blocks_per_worker + k) * B
        pltpu.sync_copy(src_hbm.at[pl.ds(off, B)], idxs)
        pltpu.sync_copy(dst_hbm.at[pl.ds(off, B)], idxd)
        cp1 = pltpu.async_copy(as_hbm.at[idxs], asr, sem1)
        cp2 = pltpu.async_copy(ad_hbm.at[idxd], adr, sem2)
        cp3 = pltpu.async_copy(ht_hbm.at[idxs], hr, sem3)
        cp1.wait()
        cp2.wait()

        def _ee(r, _):
            e = asr[r] + adr[r]
            e = jnp.maximum(e, e * 0.2)
            eeb[r] = jnp.exp(e)
            return _
        lax.fori_loop(0, B, _ee, None)
        cp3.wait()

        def _mul(r, _):
            sc = eeb[r]
            for j in range(3):
                hr[r, pl.ds(j * 16, 16)] = hr[r, pl.ds(j * 16, 16)] * sc
            return _
        lax.fori_loop(0, B, _mul, None)

        pltpu.sync_copy(hr, nsp.at[idxd], add=True)
        pltpu.sync_copy(eeb, dsp.at[idxd], add=True)
        return _
    lax.fori_loop(0, blocks_per_worker, _edge_block, None)
    plsc.subcore_barrier()

    for k in range(ZCOPIES):
        r0 = base_row + k * B
        pltpu.sync_copy(nsp.at[pl.ds(r0, B)], num_out.at[pl.ds(coff + r0, B)])
        pltpu.sync_copy(dsp.at[pl.ds(r0, B)], den_out.at[pl.ds(coff + r0, B)])


def _sc2(src_p, dst_p, AS2, AD2, H2T):
    f = pl.kernel(
        _sc2_body,
        out_type=[
            jax.ShapeDtypeStruct((2 * NPAD, C2P), jnp.float32),
            jax.ShapeDtypeStruct((2 * NPAD, 16), jnp.float32),
        ],
        mesh=_mesh(),
        compiler_params=pltpu.CompilerParams(use_tc_tiling_on_sc=False),
        scratch_types=[
            pltpu.VMEM((B,), jnp.int32),
            pltpu.VMEM((B,), jnp.int32),
            pltpu.VMEM((B, 16), jnp.float32),
            pltpu.VMEM((B, 16), jnp.float32),
            pltpu.VMEM((B, 16), jnp.float32),
            pltpu.VMEM((B, C2P), jnp.float32),
            pltpu.VMEM((B, C2P), jnp.float32),
            pltpu.VMEM((B, 16), jnp.float32),
            pltpu.VMEM_SHARED((NPAD, C2P), jnp.float32),
            pltpu.VMEM_SHARED((NPAD, 16), jnp.float32),
            pltpu.SemaphoreType.DMA,
            pltpu.SemaphoreType.DMA,
            pltpu.SemaphoreType.DMA,
        ],
    )
    return f(src_p, dst_p, AS2, AD2, H2T)


def _tc3_body(n2_ref, d2_ref, b2_ref, out_ref):
    nsum = n2_ref[0, :, :] + n2_ref[1, :, :]
    dsum = d2_ref[0, :, :1] + d2_ref[1, :, :1]
    o = nsum / (dsum + 1e-16) + b2_ref[...]
    mask = lax.broadcasted_iota(jnp.int32, o.shape, 1) < C2
    om = jnp.where(mask, o, -1e30)
    m = jnp.max(om, axis=1, keepdims=True)
    ex = jnp.where(mask, jnp.exp(o - m), 0.0)
    lse = m + jnp.log(jnp.sum(ex, axis=1, keepdims=True))
    out_ref[...] = o - lse


def _tc3(n2, d2, b2r):
    grid = (NPAD // BLK,)
    return pl.pallas_call(
        _tc3_body,
        grid=grid,
        in_specs=[
            pl.BlockSpec((2, BLK, C2P), lambda i: (0, i, 0)),
            pl.BlockSpec((2, BLK, 16), lambda i: (0, i, 0)),
            pl.BlockSpec((1, C2P), lambda i: (0, 0)),
        ],
        out_specs=pl.BlockSpec((BLK, C2P), lambda i: (i, 0)),
        out_shape=jax.ShapeDtypeStruct((NPAD, C2P), jnp.float32),
    )(n2, d2, b2r)


def kernel(x, edge_index, W1, a_src1, a_dst1, b1, W2, a_src2, a_dst2, b2):
    f32 = jnp.float32
    x_pad = jnp.zeros((NPAD, D), f32).at[:N].set(x)
    loops = jnp.arange(N, dtype=jnp.int32)
    src_p = jnp.full((EPAD,), N, jnp.int32)
    src_p = src_p.at[:E].set(edge_index[0]).at[E:EP].set(loops)
    dst_p = jnp.full((EPAD,), N, jnp.int32)
    dst_p = dst_p.at[:E].set(edge_index[1]).at[E:EP].set(loops)

    eye8 = jnp.eye(H1, dtype=f32)
    A_src1 = (eye8[:, None, :] * a_src1[:, :, None]).reshape(HC1, H1)
    A_dst1 = (eye8[:, None, :] * a_dst1[:, :, None]).reshape(HC1, H1)
    Rexp = (eye8[:, :, None] * jnp.ones((1, 1, C1), f32)).reshape(H1, HC1)
    W2p = jnp.zeros((HC1, C2P), f32).at[:, :C2].set(W2)
    asp2 = jnp.zeros((C2P, 1), f32).at[:C2, 0].set(a_src2[0])
    adp2 = jnp.zeros((C2P, 1), f32).at[:C2, 0].set(a_dst2[0])
    b1r = b1.reshape(1, HC1).astype(f32)
    b2r = jnp.zeros((1, C2P), f32).at[0, :C2].set(b2)

    HT, AS, AD = _tc1(x_pad, W1, A_src1, A_dst1)
    HTf = HT.reshape(2 * NPAD, 128)
    n1, d1 = _sc1(src_p, dst_p, AS, AD, HTf)
    n1 = n1.reshape(2, NPAD, 128)
    d1 = d1[:NPAD]

    H2T, AS2, AD2 = _tc2(n1, d1, Rexp, W2p, asp2, adp2, b1r)
    n2, d2 = _sc2(src_p, dst_p, AS2, AD2, H2T)
    out = _tc3(n2.reshape(2, NPAD, C2P), d2.reshape(2, NPAD, 16), b2r)
    return out[:N, :C2]


# speedup vs baseline: 1.3644x; 1.0456x over previous
import jax
import jax.numpy as jnp
from jax import lax
from jax.experimental import pallas as pl
from jax.experimental.pallas import tpu as pltpu
from jax.experimental.pallas import tpu_sc as plsc

N = 10000
E = 320000
D = 128
H1, C1 = 8, 32
HC1 = H1 * C1
C2 = 42
C2P = 48

NPAD = 10240
NW = 32
B = 128
EP = E + N
NBLK = 2592
EPAD = NBLK * B
ROWS_PER_TILE = NPAD // 16
ZCOPIES = ROWS_PER_TILE // B
BLK = 512


def _mesh():
    return plsc.VectorSubcoreMesh(core_axis_name="c", subcore_axis_name="s")


def _tc1_body(x_ref, w1_ref, asrc_ref, adst_ref, ht_ref, as_ref, ad_ref):
    h = jnp.dot(x_ref[...], w1_ref[...], preferred_element_type=jnp.float32)
    ht_ref[0, :, :] = h[:, :128]
    ht_ref[1, :, :] = h[:, 128:]
    als = jnp.dot(h, asrc_ref[...], preferred_element_type=jnp.float32)
    ald = jnp.dot(h, adst_ref[...], preferred_element_type=jnp.float32)
    as_ref[...] = jnp.concatenate([als, als], axis=1)
    ad_ref[...] = jnp.concatenate([ald, ald], axis=1)


def _tc1(x_pad, W1, A_src1, A_dst1):
    grid = (NPAD // BLK,)
    return pl.pallas_call(
        _tc1_body,
        grid=grid,
        in_specs=[
            pl.BlockSpec((BLK, D), lambda i: (i, 0)),
            pl.BlockSpec((D, HC1), lambda i: (0, 0)),
            pl.BlockSpec((HC1, H1), lambda i: (0, 0)),
            pl.BlockSpec((HC1, H1), lambda i: (0, 0)),
        ],
        out_specs=[
            pl.BlockSpec((2, BLK, 128), lambda i: (0, i, 0)),
            pl.BlockSpec((BLK, 16), lambda i: (i, 0)),
            pl.BlockSpec((BLK, 16), lambda i: (i, 0)),
        ],
        out_shape=[
            jax.ShapeDtypeStruct((2, NPAD, 128), jnp.float32),
            jax.ShapeDtypeStruct((NPAD, 16), jnp.float32),
            jax.ShapeDtypeStruct((NPAD, 16), jnp.float32),
        ],
    )(x_pad, W1, A_src1, A_dst1)


def _sc1_body(src_hbm, dst_hbm, as_hbm, ad_hbm, ht_hbm,
              num_out, den_out,
              idxs, idxd, gidx, asr, adr, eeb, hr,
              nsp, dsp, sem1, sem2, sem3):
    c = lax.axis_index("c")
    s = lax.axis_index("s")
    base_row = s * ROWS_PER_TILE

    def _zero_row(r, _):
        for j in range(8):
            hr[r, pl.ds(j * 16, 16)] = jnp.zeros((16,), jnp.float32)
        eeb[r] = jnp.zeros((16,), jnp.float32)
        return _
    lax.fori_loop(0, B, _zero_row, None)
    for k in range(ZCOPIES):
        pltpu.sync_copy(hr, nsp.at[pl.ds(base_row + k * B, B)])
        pltpu.sync_copy(eeb, dsp.at[pl.ds(base_row + k * B, B)])
    plsc.subcore_barrier()

    blocks_per_tile = NBLK // 16
    coff = c * NPAD

    def _edge_block(k, _):
        off = (s * blocks_per_tile + k) * B
        pltpu.sync_copy(src_hbm.at[pl.ds(off, B)], idxs)
        pltpu.sync_copy(dst_hbm.at[pl.ds(off, B)], idxd)
        for j in range(8):
            gidx[pl.ds(j * 16, 16)] = idxs[pl.ds(j * 16, 16)] + coff
        cp1 = pltpu.async_copy(as_hbm.at[idxs], asr, sem1)
        cp2 = pltpu.async_copy(ad_hbm.at[idxd], adr, sem2)
        cp3 = pltpu.async_copy(ht_hbm.at[gidx], hr, sem3)
        cp1.wait()
        cp2.wait()

        def _ee(r, _):
            e = asr[r] + adr[r]
            e = jnp.maximum(e, e * 0.2)
            eeb[r] = jnp.exp(e)
            return _
        lax.fori_loop(0, B, _ee, None)
        cp3.wait()

        def _mk_mul(hbase):
            def _mul(r, _):
                v = eeb[r]
                for j in range(8):
                    m = jnp.full((16,), v[hbase + j // 2], jnp.float32)
                    hr[r, pl.ds(j * 16, 16)] = hr[r, pl.ds(j * 16, 16)] * m
                return _
            return _mul

        @pl.when(c == 0)
        def _():
            lax.fori_loop(0, B, _mk_mul(0), None)

        @pl.when(c == 1)
        def _():
            lax.fori_loop(0, B, _mk_mul(4), None)

        pltpu.sync_copy(hr, nsp.at[idxd], add=True)
        pltpu.sync_copy(eeb, dsp.at[idxd], add=True)
        return _
    lax.fori_loop(0, blocks_per_tile, _edge_block, None)
    plsc.subcore_barrier()

    for k in range(ZCOPIES):
        r0 = base_row + k * B
        pltpu.sync_copy(nsp.at[pl.ds(r0, B)], num_out.at[pl.ds(coff + r0, B)])
        pltpu.sync_copy(dsp.at[pl.ds(r0, B)], den_out.at[pl.ds(coff + r0, B)])


def _sc1(src_p, dst_p, AS, AD, HT):
    f = pl.kernel(
        _sc1_body,
        out_type=[
            jax.ShapeDtypeStruct((2 * NPAD, 128), jnp.float32),
            jax.ShapeDtypeStruct((2 * NPAD, 16), jnp.float32),
        ],
        mesh=_mesh(),
        compiler_params=pltpu.CompilerParams(use_tc_tiling_on_sc=False),
        scratch_types=[
            pltpu.VMEM((B,), jnp.int32),
            pltpu.VMEM((B,), jnp.int32),
            pltpu.VMEM((B,), jnp.int32),
            pltpu.VMEM((B, 16), jnp.float32),
            pltpu.VMEM((B, 16), jnp.float32),
            pltpu.VMEM((B, 16), jnp.float32),
            pltpu.VMEM((B, 128), jnp.float32),
            pltpu.VMEM_SHARED((NPAD, 128), jnp.float32),
            pltpu.VMEM_SHARED((NPAD, 16), jnp.float32),
            pltpu.SemaphoreType.DMA,
            pltpu.SemaphoreType.DMA,
            pltpu.SemaphoreType.DMA,
        ],
    )
    return f(src_p, dst_p, AS, AD, HT)


def _tc2_body(n1_ref, d1_ref, rexp_ref, w2_ref, asp_ref, adp_ref, b1_ref,
              h2_ref, as2_ref, ad2_ref):
    ncat = jnp.concatenate([n1_ref[0, :, :], n1_ref[1, :, :]], axis=1)
    d8 = d1_ref[...][:, :8]
    dfull = jnp.dot(d8, rexp_ref[...], preferred_element_type=jnp.float32)
    o = ncat / (dfull + 1e-16) + b1_ref[...]
    o = jnp.maximum(o, 0.0)
    h2 = jnp.dot(o, w2_ref[...], preferred_element_type=jnp.float32)
    h2_ref[...] = h2
    als = jnp.dot(h2, asp_ref[...], preferred_element_type=jnp.float32)
    ald = jnp.dot(h2, adp_ref[...], preferred_element_type=jnp.float32)
    as2_ref[...] = jnp.broadcast_to(als, (als.shape[0], 16))
    ad2_ref[...] = jnp.broadcast_to(ald, (ald.shape[0], 16))


def _tc2(n1, d1, Rexp, W2p, asp2, adp2, b1r):
    grid = (NPAD // BLK,)
    return pl.pallas_call(
        _tc2_body,
        grid=grid,
        in_specs=[
            pl.BlockSpec((2, BLK, 128), lambda i: (0, i, 0)),
            pl.BlockSpec((BLK, 16), lambda i: (i, 0)),
            pl.BlockSpec((H1, HC1), lambda i: (0, 0)),
            pl.BlockSpec((HC1, C2P), lambda i: (0, 0)),
            pl.BlockSpec((C2P, 1), lambda i: (0, 0)),
            pl.BlockSpec((C2P, 1), lambda i: (0, 0)),
            pl.BlockSpec((1, HC1), lambda i: (0, 0)),
        ],
        out_specs=[
            pl.BlockSpec((BLK, C2P), lambda i: (i, 0)),
            pl.BlockSpec((BLK, 16), lambda i: (i, 0)),
            pl.BlockSpec((BLK, 16), lambda i: (i, 0)),
        ],
        out_shape=[
            jax.ShapeDtypeStruct((NPAD, C2P), jnp.float32),
            jax.ShapeDtypeStruct((NPAD, 16), jnp.float32),
            jax.ShapeDtypeStruct((NPAD, 16), jnp.float32),
        ],
    )(n1, d1, Rexp, W2p, asp2, adp2, b1r)


def _sc2_body(src_hbm, dst_hbm, as_hbm, ad_hbm, ht_hbm,
              num_out, den_out,
              idxs, idxd, asr, adr, eeb, hr, zb, zbd,
              nsp, dsp, sem1, sem2, sem3):
    c = lax.axis_index("c")
    s = lax.axis_index("s")
    base_row = s * ROWS_PER_TILE

    def _zero_row(r, _):
        for j in range(3):
            zb[r, pl.ds(j * 16, 16)] = jnp.zeros((16,), jnp.float32)
        zbd[r] = jnp.zeros((16,), jnp.float32)
        return _
    lax.fori_loop(0, B, _zero_row, None)
    for k in range(ZCOPIES):
        pltpu.sync_copy(zb, nsp.at[pl.ds(base_row + k * B, B)])
        pltpu.sync_copy(zbd, dsp.at[pl.ds(base_row + k * B, B)])
    plsc.subcore_barrier()

    w = s * 2 + c
    blocks_per_worker = NBLK // NW
    coff = c * NPAD

    def _edge_block(k, _):
        off = (w * blocks_per_worker + k) * B
        pltpu.sync_copy(src_hbm.at[pl.ds(off, B)], idxs)
        pltpu.sync_copy(dst_hbm.at[pl.ds(off, B)], idxd)
        cp1 = pltpu.async_copy(as_hbm.at[idxs], asr, sem1)
        cp2 = pltpu.async_copy(ad_hbm.at[idxd], adr, sem2)
        cp3 = pltpu.async_copy(ht_hbm.at[idxs], hr, sem3)
        cp1.wait()
        cp2.wait()

        def _ee(r, _):
            e = asr[r] + adr[r]
            e = jnp.maximum(e, e * 0.2)
            eeb[r] = jnp.exp(e)
            return _
        lax.fori_loop(0, B, _ee, None)
        cp3.wait()

        def _mul(r, _):
            sc = eeb[r]
            for j in range(3):
                hr[r, pl.ds(j * 16, 16)] = hr[r, pl.ds(j * 16, 16)] * sc
            return _
        lax.fori_loop(0, B, _mul, None)

        pltpu.sync_copy(hr, nsp.at[idxd], add=True)
        pltpu.sync_copy(eeb, dsp.at[idxd], add=True)
        return _
    lax.fori_loop(0, blocks_per_worker, _edge_block, None)
    plsc.subcore_barrier()

    for k in range(ZCOPIES):
        r0 = base_row + k * B
        pltpu.sync_copy(nsp.at[pl.ds(r0, B)], num_out.at[pl.ds(coff + r0, B)])
        pltpu.sync_copy(dsp.at[pl.ds(r0, B)], den_out.at[pl.ds(coff + r0, B)])


def _sc2(src_p, dst_p, AS2, AD2, H2T):
    f = pl.kernel(
        _sc2_body,
        out_type=[
            jax.ShapeDtypeStruct((2 * NPAD, C2P), jnp.float32),
            jax.ShapeDtypeStruct((2 * NPAD, 16), jnp.float32),
        ],
        mesh=_mesh(),
        compiler_params=pltpu.CompilerParams(use_tc_tiling_on_sc=False),
        scratch_types=[
            pltpu.VMEM((B,), jnp.int32),
            pltpu.VMEM((B,), jnp.int32),
            pltpu.VMEM((B, 16), jnp.float32),
            pltpu.VMEM((B, 16), jnp.float32),
            pltpu.VMEM((B, 16), jnp.float32),
            pltpu.VMEM((B, C2P), jnp.float32),
            pltpu.VMEM((B, C2P), jnp.float32),
            pltpu.VMEM((B, 16), jnp.float32),
            pltpu.VMEM_SHARED((NPAD, C2P), jnp.float32),
            pltpu.VMEM_SHARED((NPAD, 16), jnp.float32),
            pltpu.SemaphoreType.DMA,
            pltpu.SemaphoreType.DMA,
            pltpu.SemaphoreType.DMA,
        ],
    )
    return f(src_p, dst_p, AS2, AD2, H2T)


def _tc3_body(n2_ref, d2_ref, b2_ref, out_ref):
    nsum = n2_ref[0, :, :] + n2_ref[1, :, :]
    dsum = d2_ref[0, :, :1] + d2_ref[1, :, :1]
    o = nsum / (dsum + 1e-16) + b2_ref[...]
    mask = lax.broadcasted_iota(jnp.int32, o.shape, 1) < C2
    om = jnp.where(mask, o, -1e30)
    m = jnp.max(om, axis=1, keepdims=True)
    ex = jnp.where(mask, jnp.exp(o - m), 0.0)
    lse = m + jnp.log(jnp.sum(ex, axis=1, keepdims=True))
    out_ref[...] = o - lse


def _tc3(n2, d2, b2r):
    grid = (NPAD // BLK,)
    return pl.pallas_call(
        _tc3_body,
        grid=grid,
        in_specs=[
            pl.BlockSpec((2, BLK, C2P), lambda i: (0, i, 0)),
            pl.BlockSpec((2, BLK, 16), lambda i: (0, i, 0)),
            pl.BlockSpec((1, C2P), lambda i: (0, 0)),
        ],
        out_specs=pl.BlockSpec((BLK, C2P), lambda i: (i, 0)),
        out_shape=jax.ShapeDtypeStruct((NPAD, C2P), jnp.float32),
    )(n2, d2, b2r)


def kernel(x, edge_index, W1, a_src1, a_dst1, b1, W2, a_src2, a_dst2, b2):
    f32 = jnp.float32
    x_pad = jnp.zeros((NPAD, D), f32).at[:N].set(x)
    loops = jnp.arange(N, dtype=jnp.int32)
    src_p = jnp.full((EPAD,), N, jnp.int32)
    src_p = src_p.at[:E].set(edge_index[0]).at[E:EP].set(loops)
    dst_p = jnp.full((EPAD,), N, jnp.int32)
    dst_p = dst_p.at[:E].set(edge_index[1]).at[E:EP].set(loops)

    eye8 = jnp.eye(H1, dtype=f32)
    A_src1 = (eye8[:, None, :] * a_src1[:, :, None]).reshape(HC1, H1)
    A_dst1 = (eye8[:, None, :] * a_dst1[:, :, None]).reshape(HC1, H1)
    Rexp = (eye8[:, :, None] * jnp.ones((1, 1, C1), f32)).reshape(H1, HC1)
    W2p = jnp.zeros((HC1, C2P), f32).at[:, :C2].set(W2)
    asp2 = jnp.zeros((C2P, 1), f32).at[:C2, 0].set(a_src2[0])
    adp2 = jnp.zeros((C2P, 1), f32).at[:C2, 0].set(a_dst2[0])
    b1r = b1.reshape(1, HC1).astype(f32)
    b2r = jnp.zeros((1, C2P), f32).at[0, :C2].set(b2)

    HT, AS, AD = _tc1(x_pad, W1, A_src1, A_dst1)
    HTf = HT.reshape(2 * NPAD, 128)
    n1, d1 = _sc1(src_p, dst_p, AS, AD, HTf)
    n1 = n1.reshape(2, NPAD, 128)
    d1 = d1[:NPAD]

    H2T, AS2, AD2 = _tc2(n1, d1, Rexp, W2p, asp2, adp2, b1r)
    n2, d2 = _sc2(src_p, dst_p, AS2, AD2, H2T)
    out = _tc3(n2.reshape(2, NPAD, C2P), d2.reshape(2, NPAD, 16), b2r)
    return out[:N, :C2]


# speedup vs baseline: 1.4299x; 1.0480x over previous
import jax
import jax.numpy as jnp
from jax import lax
from jax.experimental import pallas as pl
from jax.experimental.pallas import tpu as pltpu
from jax.experimental.pallas import tpu_sc as plsc

N = 10000
E = 320000
D = 128
H1, C1 = 8, 32
HC1 = H1 * C1
C2 = 42
C2P = 48

NPAD = 10240
NW = 32
B = 128
EP = E + N
NBLK = 2592
EPAD = NBLK * B
ROWS_PER_TILE = NPAD // 16
ZCOPIES = ROWS_PER_TILE // B
BLK = 512


def _mesh():
    return plsc.VectorSubcoreMesh(core_axis_name="c", subcore_axis_name="s")


def _tc1_body(x_ref, w1_ref, asrc_ref, adst_ref, ht_ref, as_ref, ad_ref):
    h = jnp.dot(x_ref[...], w1_ref[...], preferred_element_type=jnp.float32)
    ht_ref[0, :, :] = h[:, :128]
    ht_ref[1, :, :] = h[:, 128:]
    als = jnp.dot(h, asrc_ref[...], preferred_element_type=jnp.float32)
    ald = jnp.dot(h, adst_ref[...], preferred_element_type=jnp.float32)
    as_ref[...] = jnp.concatenate([als, als], axis=1)
    ad_ref[...] = jnp.concatenate([ald, ald], axis=1)


def _tc1(x_pad, W1, A_src1, A_dst1):
    grid = (NPAD // BLK,)
    return pl.pallas_call(
        _tc1_body,
        grid=grid,
        in_specs=[
            pl.BlockSpec((BLK, D), lambda i: (i, 0)),
            pl.BlockSpec((D, HC1), lambda i: (0, 0)),
            pl.BlockSpec((HC1, H1), lambda i: (0, 0)),
            pl.BlockSpec((HC1, H1), lambda i: (0, 0)),
        ],
        out_specs=[
            pl.BlockSpec((2, BLK, 128), lambda i: (0, i, 0)),
            pl.BlockSpec((BLK, 16), lambda i: (i, 0)),
            pl.BlockSpec((BLK, 16), lambda i: (i, 0)),
        ],
        out_shape=[
            jax.ShapeDtypeStruct((2, NPAD, 128), jnp.float32),
            jax.ShapeDtypeStruct((NPAD, 16), jnp.float32),
            jax.ShapeDtypeStruct((NPAD, 16), jnp.float32),
        ],
    )(x_pad, W1, A_src1, A_dst1)


def _sc1_body(ei_hbm, as_hbm, ad_hbm, ht_hbm,
              num_out, den_out,
              idxsd, idxd, gidx, asr, adr, eeb, hr,
              nsp, dsp, sem1, sem2, sem3):
    c = lax.axis_index("c")
    s = lax.axis_index("s")
    base_row = s * ROWS_PER_TILE

    def _zero_row(r, _):
        for j in range(8):
            hr[r, pl.ds(j * 16, 16)] = jnp.zeros((16,), jnp.float32)
        eeb[r] = jnp.zeros((16,), jnp.float32)
        return _
    lax.fori_loop(0, B, _zero_row, None)
    for k in range(ZCOPIES):
        pltpu.sync_copy(hr, nsp.at[pl.ds(base_row + k * B, B)])
        pltpu.sync_copy(eeb, dsp.at[pl.ds(base_row + k * B, B)])
    plsc.subcore_barrier()

    blocks_per_tile = NBLK // 16
    coff = c * NPAD

    def _edge_block(k, _):
        off = (s * blocks_per_tile + k) * 2 * B
        pltpu.sync_copy(ei_hbm.at[pl.ds(off, 2 * B)], idxsd)
        for j in range(8):
            gidx[pl.ds(j * 16, 16)] = idxsd[pl.ds(j * 16, 16)] + coff
            idxd[pl.ds(j * 16, 16)] = idxsd[pl.ds(B + j * 16, 16)]
        cp1 = pltpu.async_copy(as_hbm.at[idxsd.at[pl.ds(0, B)]], asr, sem1)
        cp2 = pltpu.async_copy(ad_hbm.at[idxd], adr, sem2)
        cp3 = pltpu.async_copy(ht_hbm.at[gidx], hr, sem3)
        cp1.wait()
        cp2.wait()

        def _ee(r, _):
            e = asr[r] + adr[r]
            e = jnp.maximum(e, e * 0.2)
            eeb[r] = jnp.exp(e)
            return _
        lax.fori_loop(0, B, _ee, None)
        cp3.wait()

        def _mk_mul(hbase):
            def _mul(r, _):
                v = eeb[r]
                for j in range(8):
                    m = jnp.full((16,), v[hbase + j // 2], jnp.float32)
                    hr[r, pl.ds(j * 16, 16)] = hr[r, pl.ds(j * 16, 16)] * m
                return _
            return _mul

        @pl.when(c == 0)
        def _():
            lax.fori_loop(0, B, _mk_mul(0), None)

        @pl.when(c == 1)
        def _():
            lax.fori_loop(0, B, _mk_mul(4), None)

        pltpu.sync_copy(hr, nsp.at[idxd], add=True)
        pltpu.sync_copy(eeb, dsp.at[idxd], add=True)
        return _
    lax.fori_loop(0, blocks_per_tile, _edge_block, None)
    plsc.subcore_barrier()

    for k in range(ZCOPIES):
        r0 = base_row + k * B
        pltpu.sync_copy(nsp.at[pl.ds(r0, B)], num_out.at[pl.ds(coff + r0, B)])
        pltpu.sync_copy(dsp.at[pl.ds(r0, B)], den_out.at[pl.ds(coff + r0, B)])


def _sc1(eip, AS, AD, HT):
    f = pl.kernel(
        _sc1_body,
        out_type=[
            jax.ShapeDtypeStruct((2 * NPAD, 128), jnp.float32),
            jax.ShapeDtypeStruct((2 * NPAD, 16), jnp.float32),
        ],
        mesh=_mesh(),
        compiler_params=pltpu.CompilerParams(use_tc_tiling_on_sc=False),
        scratch_types=[
            pltpu.VMEM((2 * B,), jnp.int32),
            pltpu.VMEM((B,), jnp.int32),
            pltpu.VMEM((B,), jnp.int32),
            pltpu.VMEM((B, 16), jnp.float32),
            pltpu.VMEM((B, 16), jnp.float32),
            pltpu.VMEM((B, 16), jnp.float32),
            pltpu.VMEM((B, 128), jnp.float32),
            pltpu.VMEM_SHARED((NPAD, 128), jnp.float32),
            pltpu.VMEM_SHARED((NPAD, 16), jnp.float32),
            pltpu.SemaphoreType.DMA,
            pltpu.SemaphoreType.DMA,
            pltpu.SemaphoreType.DMA,
        ],
    )
    return f(eip, AS, AD, HT)


def _tc2_body(n1_ref, d1_ref, rexp_ref, w2_ref, asp_ref, adp_ref, b1_ref,
              h2_ref, as2_ref, ad2_ref):
    ncat = jnp.concatenate([n1_ref[0, :, :], n1_ref[1, :, :]], axis=1)
    d8 = d1_ref[...][:, :8]
    dfull = jnp.dot(d8, rexp_ref[...], preferred_element_type=jnp.float32)
    o = ncat / (dfull + 1e-16) + b1_ref[...]
    o = jnp.maximum(o, 0.0)
    h2 = jnp.dot(o, w2_ref[...], preferred_element_type=jnp.float32)
    h2_ref[...] = h2
    als = jnp.dot(h2, asp_ref[...], preferred_element_type=jnp.float32)
    ald = jnp.dot(h2, adp_ref[...], preferred_element_type=jnp.float32)
    as2_ref[...] = jnp.broadcast_to(als, (als.shape[0], 16))
    ad2_ref[...] = jnp.broadcast_to(ald, (ald.shape[0], 16))


def _tc2(n1, d1, Rexp, W2p, asp2, adp2, b1r):
    grid = (NPAD // BLK,)
    return pl.pallas_call(
        _tc2_body,
        grid=grid,
        in_specs=[
            pl.BlockSpec((2, BLK, 128), lambda i: (0, i, 0)),
            pl.BlockSpec((BLK, 16), lambda i: (i, 0)),
            pl.BlockSpec((H1, HC1), lambda i: (0, 0)),
            pl.BlockSpec((HC1, C2P), lambda i: (0, 0)),
            pl.BlockSpec((C2P, 1), lambda i: (0, 0)),
            pl.BlockSpec((C2P, 1), lambda i: (0, 0)),
            pl.BlockSpec((1, HC1), lambda i: (0, 0)),
        ],
        out_specs=[
            pl.BlockSpec((BLK, C2P), lambda i: (i, 0)),
            pl.BlockSpec((BLK, 16), lambda i: (i, 0)),
            pl.BlockSpec((BLK, 16), lambda i: (i, 0)),
        ],
        out_shape=[
            jax.ShapeDtypeStruct((NPAD, C2P), jnp.float32),
            jax.ShapeDtypeStruct((NPAD, 16), jnp.float32),
            jax.ShapeDtypeStruct((NPAD, 16), jnp.float32),
        ],
    )(n1, d1, Rexp, W2p, asp2, adp2, b1r)


def _sc2_body(ei_hbm, as_hbm, ad_hbm, ht_hbm,
              num_out, den_out,
              idxsd, idxd, asr, adr, eeb, hr, zb, zbd,
              nsp, dsp, sem1, sem2, sem3):
    c = lax.axis_index("c")
    s = lax.axis_index("s")
    base_row = s * ROWS_PER_TILE

    def _zero_row(r, _):
        for j in range(3):
            zb[r, pl.ds(j * 16, 16)] = jnp.zeros((16,), jnp.float32)
        zbd[r] = jnp.zeros((16,), jnp.float32)
        return _
    lax.fori_loop(0, B, _zero_row, None)
    for k in range(ZCOPIES):
        pltpu.sync_copy(zb, nsp.at[pl.ds(base_row + k * B, B)])
        pltpu.sync_copy(zbd, dsp.at[pl.ds(base_row + k * B, B)])
    plsc.subcore_barrier()

    w = s * 2 + c
    blocks_per_worker = NBLK // NW
    coff = c * NPAD

    def _edge_block(k, _):
        off = (w * blocks_per_worker + k) * 2 * B
        pltpu.sync_copy(ei_hbm.at[pl.ds(off, 2 * B)], idxsd)
        for j in range(8):
            idxd[pl.ds(j * 16, 16)] = idxsd[pl.ds(B + j * 16, 16)]
        cp1 = pltpu.async_copy(as_hbm.at[idxsd.at[pl.ds(0, B)]], asr, sem1)
        cp2 = pltpu.async_copy(ad_hbm.at[idxd], adr, sem2)
        cp3 = pltpu.async_copy(ht_hbm.at[idxsd.at[pl.ds(0, B)]], hr, sem3)
        cp1.wait()
        cp2.wait()

        def _ee(r, _):
            e = asr[r] + adr[r]
            e = jnp.maximum(e, e * 0.2)
            eeb[r] = jnp.exp(e)
            return _
        lax.fori_loop(0, B, _ee, None)
        cp3.wait()

        def _mul(r, _):
            sc = eeb[r]
            for j in range(3):
                hr[r, pl.ds(j * 16, 16)] = hr[r, pl.ds(j * 16, 16)] * sc
            return _
        lax.fori_loop(0, B, _mul, None)

        pltpu.sync_copy(hr, nsp.at[idxd], add=True)
        pltpu.sync_copy(eeb, dsp.at[idxd], add=True)
        return _
    lax.fori_loop(0, blocks_per_worker, _edge_block, None)
    plsc.subcore_barrier()

    for k in range(ZCOPIES):
        r0 = base_row + k * B
        pltpu.sync_copy(nsp.at[pl.ds(r0, B)], num_out.at[pl.ds(coff + r0, B)])
        pltpu.sync_copy(dsp.at[pl.ds(r0, B)], den_out.at[pl.ds(coff + r0, B)])


def _sc2(eip, AS2, AD2, H2T):
    f = pl.kernel(
        _sc2_body,
        out_type=[
            jax.ShapeDtypeStruct((2 * NPAD, C2P), jnp.float32),
            jax.ShapeDtypeStruct((2 * NPAD, 16), jnp.float32),
        ],
        mesh=_mesh(),
        compiler_params=pltpu.CompilerParams(use_tc_tiling_on_sc=False),
        scratch_types=[
            pltpu.VMEM((2 * B,), jnp.int32),
            pltpu.VMEM((B,), jnp.int32),
            pltpu.VMEM((B, 16), jnp.float32),
            pltpu.VMEM((B, 16), jnp.float32),
            pltpu.VMEM((B, 16), jnp.float32),
            pltpu.VMEM((B, C2P), jnp.float32),
            pltpu.VMEM((B, C2P), jnp.float32),
            pltpu.VMEM((B, 16), jnp.float32),
            pltpu.VMEM_SHARED((NPAD, C2P), jnp.float32),
            pltpu.VMEM_SHARED((NPAD, 16), jnp.float32),
            pltpu.SemaphoreType.DMA,
            pltpu.SemaphoreType.DMA,
            pltpu.SemaphoreType.DMA,
        ],
    )
    return f(eip, AS2, AD2, H2T)


def _tc3_body(n2_ref, d2_ref, b2_ref, out_ref):
    nsum = n2_ref[0, :, :] + n2_ref[1, :, :]
    dsum = d2_ref[0, :, :1] + d2_ref[1, :, :1]
    o = nsum / (dsum + 1e-16) + b2_ref[...]
    mask = lax.broadcasted_iota(jnp.int32, o.shape, 1) < C2
    om = jnp.where(mask, o, -1e30)
    m = jnp.max(om, axis=1, keepdims=True)
    ex = jnp.where(mask, jnp.exp(o - m), 0.0)
    lse = m + jnp.log(jnp.sum(ex, axis=1, keepdims=True))
    out_ref[...] = o - lse


def _tc3(n2, d2, b2r):
    grid = (NPAD // BLK,)
    return pl.pallas_call(
        _tc3_body,
        grid=grid,
        in_specs=[
            pl.BlockSpec((2, BLK, C2P), lambda i: (0, i, 0)),
            pl.BlockSpec((2, BLK, 16), lambda i: (0, i, 0)),
            pl.BlockSpec((1, C2P), lambda i: (0, 0)),
        ],
        out_specs=pl.BlockSpec((BLK, C2P), lambda i: (i, 0)),
        out_shape=jax.ShapeDtypeStruct((NPAD, C2P), jnp.float32),
    )(n2, d2, b2r)


def kernel(x, edge_index, W1, a_src1, a_dst1, b1, W2, a_src2, a_dst2, b2):
    f32 = jnp.float32
    x_pad = jnp.zeros((NPAD, D), f32).at[:N].set(x)
    loops = jnp.arange(N, dtype=jnp.int32)
    src_p = jnp.full((EPAD,), N, jnp.int32)
    src_p = src_p.at[:E].set(edge_index[0]).at[E:EP].set(loops)
    dst_p = jnp.full((EPAD,), N, jnp.int32)
    dst_p = dst_p.at[:E].set(edge_index[1]).at[E:EP].set(loops)
    eip = jnp.stack([src_p.reshape(NBLK, B), dst_p.reshape(NBLK, B)],
                    axis=1).reshape(NBLK * 2 * B)

    eye8 = jnp.eye(H1, dtype=f32)
    A_src1 = (eye8[:, None, :] * a_src1[:, :, None]).reshape(HC1, H1)
    A_dst1 = (eye8[:, None, :] * a_dst1[:, :, None]).reshape(HC1, H1)
    Rexp = (eye8[:, :, None] * jnp.ones((1, 1, C1), f32)).reshape(H1, HC1)
    W2p = jnp.zeros((HC1, C2P), f32).at[:, :C2].set(W2)
    asp2 = jnp.zeros((C2P, 1), f32).at[:C2, 0].set(a_src2[0])
    adp2 = jnp.zeros((C2P, 1), f32).at[:C2, 0].set(a_dst2[0])
    b1r = b1.reshape(1, HC1).astype(f32)
    b2r = jnp.zeros((1, C2P), f32).at[0, :C2].set(b2)

    HT, AS, AD = _tc1(x_pad, W1, A_src1, A_dst1)
    HTf = HT.reshape(2 * NPAD, 128)
    n1, d1 = _sc1(eip, AS, AD, HTf)
    n1 = n1.reshape(2, NPAD, 128)
    d1 = d1[:NPAD]

    H2T, AS2, AD2 = _tc2(n1, d1, Rexp, W2p, asp2, adp2, b1r)
    n2, d2 = _sc2(eip, AS2, AD2, H2T)
    out = _tc3(n2.reshape(2, NPAD, C2P), d2.reshape(2, NPAD, 16), b2r)
    return out[:N, :C2]
